# Initial kernel scaffold; baseline (speedup 1.0000x reference)
import jax
import jax.numpy as jnp
from jax.experimental import pallas as pl


def kernel(x, edge_index, segment_ids, Ws_self, Ws_nbr, bs, Wa_self, Wa_nbr, ba, H1, b1, H2, b2, H3, b3, H4, b4, H5, b5):
    raise NotImplementedError("write your pallas kernel here")


# speedup vs baseline: 6.0626x; 6.0626x over previous
import jax
import jax.numpy as jnp
from jax import lax
from jax.experimental import pallas as pl
from jax.experimental.pallas import tpu as pltpu
from jax.experimental.pallas import tpu_sc as plsc

N = 10000
D = 128
E = 320000
G = 100
GP = 104
NC = 2
NS = 16
HN = N // NC
NG = 32
AR = HN + NG + 8
W = 80
EPT = E // NS
WINS = EPT // W
Z0 = 312
ZL = AR - (NS - 1) * Z0
D0 = 312
DL = HN - (NS - 1) * D0
PW = 80
PWINS = N // PW

_mesh = plsc.VectorSubcoreMesh(
    core_axis_name="c", subcore_axis_name="s", num_cores=NC, num_subcores=NS)


def _edge_agg_body(K, *refs):
    (src_hbm, dst_hbm, zeros_hbm), rest = refs[:3], refs[3:]
    tables = rest[:K]
    outs = rest[K:2 * K]
    srcv, dstv, idxm, rowb, acc, sem = rest[2 * K:]
    c = lax.axis_index("c")
    s = lax.axis_index("s")
    base = c * HN
    pltpu.sync_copy(src_hbm.at[s], srcv)
    pltpu.sync_copy(dst_hbm.at[s], dstv)
    lanes = lax.iota(jnp.int32, 16)

    def remap(w, _):
        for j in range(W // 16):
            v = dstv[w, pl.ds(j * 16, 16)]
            rel = v - base
            inh = (rel >= 0) & (rel < HN)
            garb = HN + ((v + lanes) & (NG - 1))
            dstv[w, pl.ds(j * 16, 16)] = jnp.where(inh, rel, garb)
        return 0

    lax.fori_loop(0, WINS, remap, 0)
    zoff = pl.multiple_of(s * Z0, 8)
    doff = pl.multiple_of(s * D0, 8)
    for k in range(K):
        @pl.when(s < NS - 1)
        def _():
            pltpu.sync_copy(zeros_hbm.at[pl.ds(0, Z0)], acc.at[pl.ds(zoff, Z0)])

        @pl.when(s == NS - 1)
        def _():
            pltpu.sync_copy(zeros_hbm, acc.at[pl.ds((NS - 1) * Z0, ZL)])

        plsc.subcore_barrier()
        table = tables[k]
        pltpu.async_copy(table.at[srcv.at[0]], rowb.at[0], sem.at[0])

        def step(w, _):
            pltpu.async_copy(table.at[srcv.at[w + 1]],
                             rowb.at[lax.rem(w + 1, 2)],
                             sem.at[lax.rem(w + 1, 2)])
            pltpu.make_async_copy(
                table.at[srcv.at[w]], rowb.at[lax.rem(w, 2)],
                sem.at[lax.rem(w, 2)]).wait()
            pltpu.sync_copy(rowb.at[lax.rem(w, 2)], acc.at[dstv.at[w]],
                            add=True)
            return 0

        lax.fori_loop(0, WINS - 1, step, 0)
        last = WINS - 1
        pltpu.make_async_copy(
            table.at[srcv.at[last]], rowb.at[lax.rem(last, 2)],
            sem.at[lax.rem(last, 2)]).wait()
        pltpu.sync_copy(rowb.at[lax.rem(last, 2)], acc.at[dstv.at[last]],
                        add=True)
        plsc.subcore_barrier()

        @pl.when(s < NS - 1)
        def _():
            pltpu.sync_copy(acc.at[pl.ds(doff, D0)],
                            outs[k].at[pl.ds(base + doff, D0)])

        @pl.when(s == NS - 1)
        def _():
            pltpu.sync_copy(acc.at[pl.ds((NS - 1) * D0, DL)],
                            outs[k].at[pl.ds(base + (NS - 1) * D0, DL)])

        if k + 1 < K:
            plsc.subcore_barrier()


def _make_edge_agg(K):
    def body(*refs):
        _edge_agg_body(K, *refs)
    return pl.kernel(
        body,
        out_type=[jax.ShapeDtypeStruct((N, D), jnp.float32)
                  for _ in range(K)],
        mesh=_mesh,
        scratch_types=[
            pltpu.VMEM((WINS, W), jnp.int32),
            pltpu.VMEM((WINS, W), jnp.int32),
            pltpu.VMEM((W,), jnp.int32),
            pltpu.VMEM((2, W, D), jnp.float32),
            pltpu.VMEM_SHARED((AR, D), jnp.float32),
            pltpu.SemaphoreType.DMA((2,)),
        ],
    )


def _pool_body(seg_hbm, zeros3_hbm, ones_hbm, lat3_hbm, cur3_hbm, out_hbm,
               segv, rowb, onesb, accb, accc, accn, sem):
    c = lax.axis_index("c")
    s = lax.axis_index("s")
    wid = c * NS + s
    pltpu.sync_copy(ones_hbm, onesb)
    pltpu.sync_copy(seg_hbm, segv)

    @pl.when(s == 0)
    def _():
        pltpu.sync_copy(zeros3_hbm.at[0], accb)
        pltpu.sync_copy(zeros3_hbm.at[1], accc)
        pltpu.sync_copy(zeros3_hbm.at[2], accn)

    plsc.subcore_barrier()

    def do_window(j):
        pltpu.sync_copy(lat3_hbm.at[pl.ds(j * PW, PW)], rowb)
        pltpu.sync_copy(rowb, accb.at[segv.at[j]], add=True)
        pltpu.sync_copy(cur3_hbm.at[pl.ds(j * PW, PW)], rowb)
        pltpu.sync_copy(rowb, accc.at[segv.at[j]], add=True)
        pltpu.sync_copy(onesb, accn.at[segv.at[j]], add=True)

    per = PWINS // NW_POOL
    npre = per * NW_POOL
    for t in range(per):
        do_window(wid * per + t)

    @pl.when(wid < PWINS - npre)
    def _():
        do_window(npre + wid)

    plsc.subcore_barrier()

    @pl.when(s == 0)
    def _():
        for cc in range(NC):
            @pl.when(c == cc)
            def _(cc=cc):
                pltpu.sync_copy(accb, out_hbm.at[0, cc])
                pltpu.sync_copy(accc, out_hbm.at[1, cc])
                pltpu.sync_copy(accn, out_hbm.at[2, cc])


NW_POOL = NC * NS

_pool_kernel = pl.kernel(
    _pool_body,
    out_type=jax.ShapeDtypeStruct((3, NC, GP, D), jnp.float32),
    mesh=_mesh,
    scratch_types=[
        pltpu.VMEM((PWINS, PW), jnp.int32),
        pltpu.VMEM((PW, D), jnp.float32),
        pltpu.VMEM((PW, D), jnp.float32),
        pltpu.VMEM_SHARED((GP, D), jnp.float32),
        pltpu.VMEM_SHARED((GP, D), jnp.float32),
        pltpu.VMEM_SHARED((GP, D), jnp.float32),
        pltpu.SemaphoreType.DMA,
    ],
)


RB = 2000
GRID = N // RB


def _f32dot(a, b):
    return jnp.dot(a, b, preferred_element_type=jnp.float32)


def _single_body(x, a, w1, w2, b, o):
    o[...] = jnp.maximum(
        _f32dot(x[...], w1[...]) + _f32dot(a[...], w2[...]) + b[...], 0.0)


def _dual_body(p, q, ap, aq, w1, w2, w3, w4, w5, w6, b1, b2, o1, o2):
    o1[...] = jnp.maximum(
        _f32dot(p[...], w1[...]) + _f32dot(ap[...], w2[...]) + b1[...], 0.0)
    o2[...] = jnp.maximum(
        _f32dot(p[...], w3[...]) + _f32dot(q[...], w4[...])
        + _f32dot(ap[...], w5[...]) + _f32dot(aq[...], w6[...]) + b2[...],
        0.0)


def _quad_body(p, q, ap, aq, w3, w4, w5, w6, b2, o2):
    o2[...] = jnp.maximum(
        _f32dot(p[...], w3[...]) + _f32dot(q[...], w4[...])
        + _f32dot(ap[...], w5[...]) + _f32dot(aq[...], w6[...]) + b2[...],
        0.0)


def _row_spec():
    return pl.BlockSpec((RB, D), lambda i: (i, 0))


def _w_spec():
    return pl.BlockSpec((D, D), lambda i: (0, 0))


def _b_spec():
    return pl.BlockSpec((1, D), lambda i: (0, 0))


def _nd():
    return jax.ShapeDtypeStruct((N, D), jnp.float32)


def _tc_single(x, a, w1, w2, b):
    return pl.pallas_call(
        _single_body,
        grid=(GRID,),
        in_specs=[_row_spec()] * 2 + [_w_spec()] * 2 + [_b_spec()],
        out_specs=_row_spec(),
        out_shape=_nd(),
    )(x, a, w1, w2, b)


def _tc_dual(p, q, ap, aq, w1, w2, w3, w4, w5, w6, b1, b2):
    return pl.pallas_call(
        _dual_body,
        grid=(GRID,),
        in_specs=[_row_spec()] * 4 + [_w_spec()] * 6 + [_b_spec()] * 2,
        out_specs=[_row_spec()] * 2,
        out_shape=[_nd(), _nd()],
    )(p, q, ap, aq, w1, w2, w3, w4, w5, w6, b1, b2)


def _tc_quad(p, q, ap, aq, w3, w4, w5, w6, b2):
    return pl.pallas_call(
        _quad_body,
        grid=(GRID,),
        in_specs=[_row_spec()] * 4 + [_w_spec()] * 4 + [_b_spec()],
        out_specs=_row_spec(),
        out_shape=_nd(),
    )(p, q, ap, aq, w3, w4, w5, w6, b2)


def _head_body(pool, h1a, h1b, h2, h3, h4, h5, b1, b2, b3, b4, b5, o):
    cnt = jnp.maximum(pool[2, 0] + pool[2, 1], 1.0)
    base = (pool[0, 0] + pool[0, 1]) / cnt
    adapt = (pool[1, 0] + pool[1, 1]) / cnt
    t1 = _f32dot(base, h1a[...]) + _f32dot(adapt, h1b[...]) + b1[...]
    t2 = jnp.maximum(_f32dot(t1, h2[...]) + b2[...], 0.0)
    t3 = _f32dot(t2, h3[...]) + b3[...]
    t4 = jnp.maximum(_f32dot(t3, h4[...]) + b4[...], 0.0)
    o[...] = _f32dot(t4, h5[...]) + b5[...]


def _tc_head(pool, h1a, h1b, h2, h3, h4, h5p, b1, b2, b3, b4, b5p):
    return pl.pallas_call(
        _head_body,
        out_shape=jax.ShapeDtypeStruct((GP, 8), jnp.float32),
    )(pool, h1a, h1b, h2, h3, h4, h5p, b1, b2, b3, b4, b5p)


def kernel(x, edge_index, segment_ids, Ws_self, Ws_nbr, bs, Wa_self, Wa_nbr,
           ba, H1, b1, H2, b2, H3, b3, H4, b4, H5, b5):
    src_r = edge_index[0].reshape(NS, WINS, W)
    dst_r = edge_index[1].reshape(NS, WINS, W)
    seg_r = segment_ids.reshape(PWINS, PW)
    zeros = jnp.zeros((ZL, D), jnp.float32)
    zeros3 = jnp.zeros((3, GP, D), jnp.float32)
    ones = jnp.ones((PW, D), jnp.float32)

    agg1 = _make_edge_agg(1)
    agg2 = _make_edge_agg(2)

    bs_ = [bs[i][None] for i in range(3)]
    ba_ = [ba[i][None] for i in range(3)]
    WsS = [Ws_self[i] for i in range(3)]
    WsN = [Ws_nbr[i] for i in range(3)]
    WaSa = [Wa_self[i][:D] for i in range(3)]
    WaSb = [Wa_self[i][D:] for i in range(3)]
    WaNa = [Wa_nbr[i][:D] for i in range(3)]
    WaNb = [Wa_nbr[i][D:] for i in range(3)]

    (a0,) = agg1(src_r, dst_r, zeros, x)
    lat1 = _tc_single(x, a0, WsS[0], WsN[0], bs_[0])
    (a1,) = agg1(src_r, dst_r, zeros, lat1)
    lat2, cur1 = _tc_dual(lat1, x, a1, a0,
                          WsS[1], WsN[1], WaSa[0], WaSb[0], WaNa[0], WaNb[0],
                          bs_[1], ba_[0])
    a2, c1 = agg2(src_r, dst_r, zeros, lat2, cur1)
    lat3, cur2 = _tc_dual(lat2, cur1, a2, c1,
                          WsS[2], WsN[2], WaSa[1], WaSb[1], WaNa[1], WaNb[1],
                          bs_[2], ba_[1])
    a3, c2 = agg2(src_r, dst_r, zeros, lat3, cur2)
    cur3 = _tc_quad(lat3, cur2, a3, c2,
                    WaSa[2], WaSb[2], WaNa[2], WaNb[2], ba_[2])
    pool = _pool_kernel(seg_r, zeros3, ones, lat3, cur3)
    H1a, H1b = H1[:D], H1[D:]
    H5p = jnp.pad(H5, ((0, 0), (0, 7)))
    b5p = jnp.pad(b5, (0, 7))[None]
    out = _tc_head(pool, H1a, H1b, H2, H3, H4, H5p,
                   b1[None], b2[None], b3[None], b4[None], b5p)
    return out[:G, :1]
